# baseline (speedup 1.0000x reference)
import functools

import jax
import jax.numpy as jnp
from jax import lax
from jax.experimental import pallas as pl
from jax.experimental.pallas import tpu as pltpu, tpu_sc as plsc

N = 10000
E = 320000
F_IN = 128
H = 64

NC = 2
NS = 16
NW = NC * NS
CHUNK = 128
EPW = E // NW
NFC = EPW // CHUNK
REM = EPW - NFC * CHUNK
NP = 10112
RPT = NP // NS
NBUF = 6
LOOKAHEAD = NBUF - 2
CW = 8


def _mm2_body(x_ref, wl_ref, wr_ref, y_ref, z_ref):
    xb = x_ref[...]
    dn = (((1,), (1,)), ((), ()))
    y_ref[...] = lax.dot_general(xb, wl_ref[...], dn,
                                 preferred_element_type=jnp.float32)
    z_ref[...] = lax.dot_general(xb, wr_ref[...], dn,
                                 preferred_element_type=jnp.float32)


def _dual_matmul(x, wl, wr):
    blk = 1000
    grid = N // blk
    return pl.pallas_call(
        _mm2_body,
        grid=(grid,),
        in_specs=[
            pl.BlockSpec((blk, F_IN), lambda i: (i, 0)),
            pl.BlockSpec((H, F_IN), lambda i: (0, 0)),
            pl.BlockSpec((H, F_IN), lambda i: (0, 0)),
        ],
        out_specs=[
            pl.BlockSpec((blk, H), lambda i: (i, 0)),
            pl.BlockSpec((blk, H), lambda i: (i, 0)),
        ],
        out_shape=[
            jax.ShapeDtypeStruct((NP, H), jnp.float32),
            jax.ShapeDtypeStruct((NP, H), jnp.float32),
        ],
    )(x, wl, wr)


def _sc_body(y_hbm, edge_hbm, za_hbm, zc_hbm, ones_hbm,
             agg_out, cnt_out,
             agg_sh, cnt_sh,
             src_v, dst_v, rows_v, ones_v, rows16_v,
             gsem, ssem, csem):
    c = lax.axis_index("c")
    s = lax.axis_index("s")
    w = c * NS + s

    pltpu.sync_copy(za_hbm.at[pl.ds(s * RPT, RPT)], agg_sh.at[pl.ds(s * RPT, RPT)])
    pltpu.sync_copy(zc_hbm.at[pl.ds(s * RPT, RPT)], cnt_sh.at[pl.ds(s * RPT, RPT)])

    pltpu.sync_copy(edge_hbm.at[0].at[pl.ds(w * EPW, EPW)], src_v)
    pltpu.sync_copy(edge_hbm.at[1].at[pl.ds(w * EPW, EPW)], dst_v)
    pltpu.sync_copy(ones_hbm, ones_v)

    plsc.subcore_barrier()

    def sidx(j):
        return src_v.at[pl.ds(j * CHUNK, CHUNK)]

    def didx(j):
        return dst_v.at[pl.ds(j * CHUNK, CHUNK)]

    for p in range(LOOKAHEAD):
        pltpu.async_copy(y_hbm.at[sidx(p)], rows_v.at[p], gsem)

    def chunk_step(j, carry):
        b = lax.rem(j, NBUF)

        pltpu.make_async_copy(y_hbm.at[sidx(j)], rows_v.at[b], gsem).wait()

        @pl.when(j >= 2)
        def _wait_prev_scatter():
            pltpu.make_async_copy(rows_v.at[b], agg_sh.at[didx(j)],
                                  ssem).wait()
            pltpu.make_async_copy(ones_v, cnt_sh.at[didx(j)], csem).wait()

        @pl.when(j + LOOKAHEAD < NFC)
        def _prefetch_next():
            pltpu.async_copy(y_hbm.at[sidx(j + LOOKAHEAD)],
                             rows_v.at[lax.rem(j + LOOKAHEAD, NBUF)], gsem)

        pltpu.async_copy(rows_v.at[b], agg_sh.at[didx(j)], ssem, add=True)
        pltpu.async_copy(ones_v, cnt_sh.at[didx(j)], csem, add=True)
        return carry

    lax.fori_loop(0, NFC, chunk_step, 0)

    for t in (NFC - 2, NFC - 1):
        pltpu.make_async_copy(rows_v.at[lax.rem(t, NBUF)],
                              agg_sh.at[didx(t)], ssem).wait()
        pltpu.make_async_copy(ones_v, cnt_sh.at[didx(t)], csem).wait()

    rs = src_v.at[pl.ds(NFC * CHUNK, REM)]
    rd = dst_v.at[pl.ds(NFC * CHUNK, REM)]
    pltpu.sync_copy(y_hbm.at[rs], rows16_v)
    pltpu.sync_copy(rows16_v, agg_sh.at[rd], add=True)
    pltpu.sync_copy(ones_v.at[pl.ds(0, REM)], cnt_sh.at[rd], add=True)

    plsc.subcore_barrier()

    pltpu.sync_copy(agg_sh.at[pl.ds(s * RPT, RPT)],
                    agg_out.at[c].at[pl.ds(s * RPT, RPT)])
    pltpu.sync_copy(cnt_sh.at[pl.ds(s * RPT, RPT)],
                    cnt_out.at[c].at[pl.ds(s * RPT, RPT)])


_sc_aggregate = pl.kernel(
    _sc_body,
    out_type=[
        jax.ShapeDtypeStruct((NC, NP, H), jnp.float32),
        jax.ShapeDtypeStruct((NC, NP, CW), jnp.float32),
    ],
    mesh=plsc.VectorSubcoreMesh(core_axis_name="c", subcore_axis_name="s"),
    compiler_params=pltpu.CompilerParams(use_tc_tiling_on_sc=False),
    scratch_types=[
        pltpu.VMEM_SHARED((NP, H), jnp.float32),
        pltpu.VMEM_SHARED((NP, CW), jnp.float32),
        pltpu.VMEM((EPW,), jnp.int32),
        pltpu.VMEM((EPW,), jnp.int32),
        pltpu.VMEM((NBUF, CHUNK, H), jnp.float32),
        pltpu.VMEM((CHUNK, CW), jnp.float32),
        pltpu.VMEM((REM, H), jnp.float32),
        pltpu.SemaphoreType.DMA,
        pltpu.SemaphoreType.DMA,
        pltpu.SemaphoreType.DMA,
    ],
)


def _fin_body(agg_ref, cnt_ref, z_ref, b_ref, o_ref):
    a = agg_ref[0] + agg_ref[1]
    cnt = jnp.maximum(cnt_ref[0][:, :1] + cnt_ref[1][:, :1], 1.0)
    o_ref[...] = a / cnt + z_ref[...] + b_ref[...]


def _finish(agg_p, cnt_p, z, b_row):
    blk = 1000
    grid = N // blk
    return pl.pallas_call(
        _fin_body,
        grid=(grid,),
        in_specs=[
            pl.BlockSpec((NC, blk, H), lambda i: (0, i, 0)),
            pl.BlockSpec((NC, blk, CW), lambda i: (0, i, 0)),
            pl.BlockSpec((blk, H), lambda i: (i, 0)),
            pl.BlockSpec((1, H), lambda i: (0, 0)),
        ],
        out_specs=pl.BlockSpec((blk, H), lambda i: (i, 0)),
        out_shape=jax.ShapeDtypeStruct((N, H), jnp.float32),
    )(agg_p, cnt_p, z, b_row)


def kernel(x, edge_index, W_l, W_r, b_l):
    y, z = _dual_matmul(x, W_l, W_r)

    za = jnp.zeros((NP, H), jnp.float32)
    zc = jnp.zeros((NP, CW), jnp.float32)
    ones = jnp.ones((CHUNK, CW), jnp.float32)

    agg_p, cnt_p = _sc_aggregate(y, edge_index.astype(jnp.int32), za, zc, ones)

    return _finish(agg_p, cnt_p, z, b_l.reshape(1, H))


# speedup vs baseline: 1.0487x; 1.0487x over previous
import functools

import jax
import jax.numpy as jnp
from jax import lax
from jax.experimental import pallas as pl
from jax.experimental.pallas import tpu as pltpu, tpu_sc as plsc

N = 10000
E = 320000
F_IN = 128
H = 64

NC = 2
NS = 16
NW = NC * NS
CHUNK = 128
EPW = E // NW
NFC = EPW // CHUNK
REM = EPW - NFC * CHUNK
NP = 10112
RPT = NP // NS
NBUF = 6
LOOKAHEAD = NBUF - 2
CW = 16


def _mm2_body(x_ref, wl_ref, wr_ref, b_ref, y_ref, z_ref):
    xb = x_ref[...]
    dn = (((1,), (1,)), ((), ()))
    y_ref[...] = lax.dot_general(xb, wl_ref[...], dn,
                                 preferred_element_type=jnp.float32)
    z_ref[...] = lax.dot_general(xb, wr_ref[...], dn,
                                 preferred_element_type=jnp.float32) + b_ref[...]


def _dual_matmul(x, wl, wr, b_row):
    blk = 1000
    grid = N // blk
    return pl.pallas_call(
        _mm2_body,
        grid=(grid,),
        in_specs=[
            pl.BlockSpec((blk, F_IN), lambda i: (i, 0)),
            pl.BlockSpec((H, F_IN), lambda i: (0, 0)),
            pl.BlockSpec((H, F_IN), lambda i: (0, 0)),
            pl.BlockSpec((1, H), lambda i: (0, 0)),
        ],
        out_specs=[
            pl.BlockSpec((blk, H), lambda i: (i, 0)),
            pl.BlockSpec((blk, H), lambda i: (i, 0)),
        ],
        out_shape=[
            jax.ShapeDtypeStruct((NP, H), jnp.float32),
            jax.ShapeDtypeStruct((NP, H), jnp.float32),
        ],
    )(x, wl, wr, b_row)


def _sc_body(y_hbm, edge_hbm, za_hbm, zc_hbm, ones_hbm,
             agg_out, cnt_out,
             agg_sh, cnt_sh,
             src_v, dst_v, rows_v, ones_v, rows16_v,
             gsem, ssem, csem):
    c = lax.axis_index("c")
    s = lax.axis_index("s")
    w = c * NS + s

    pltpu.sync_copy(za_hbm.at[pl.ds(s * RPT, RPT)], agg_sh.at[pl.ds(s * RPT, RPT)])
    pltpu.sync_copy(zc_hbm.at[pl.ds(s * RPT, RPT)], cnt_sh.at[pl.ds(s * RPT, RPT)])

    pltpu.sync_copy(edge_hbm.at[0].at[pl.ds(w * EPW, EPW)], src_v)
    pltpu.sync_copy(edge_hbm.at[1].at[pl.ds(w * EPW, EPW)], dst_v)
    pltpu.sync_copy(ones_hbm, ones_v)

    plsc.subcore_barrier()

    def sidx(j):
        return src_v.at[pl.ds(j * CHUNK, CHUNK)]

    def didx(j):
        return dst_v.at[pl.ds(j * CHUNK, CHUNK)]

    for p in range(LOOKAHEAD):
        pltpu.async_copy(y_hbm.at[sidx(p)], rows_v.at[p], gsem)

    def chunk_step(j, carry):
        b = lax.rem(j, NBUF)

        pltpu.make_async_copy(y_hbm.at[sidx(j)], rows_v.at[b], gsem).wait()

        @pl.when(j >= 2)
        def _wait_prev_scatter():
            pltpu.make_async_copy(rows_v.at[b], agg_sh.at[didx(j)],
                                  ssem).wait()
            pltpu.make_async_copy(ones_v, cnt_sh.at[didx(j)], csem).wait()

        @pl.when(j + LOOKAHEAD < NFC)
        def _prefetch_next():
            pltpu.async_copy(y_hbm.at[sidx(j + LOOKAHEAD)],
                             rows_v.at[lax.rem(j + LOOKAHEAD, NBUF)], gsem)

        pltpu.async_copy(rows_v.at[b], agg_sh.at[didx(j)], ssem, add=True)
        pltpu.async_copy(ones_v, cnt_sh.at[didx(j)], csem, add=True)
        return carry

    lax.fori_loop(0, NFC, chunk_step, 0)

    for t in (NFC - 2, NFC - 1):
        pltpu.make_async_copy(rows_v.at[lax.rem(t, NBUF)],
                              agg_sh.at[didx(t)], ssem).wait()
        pltpu.make_async_copy(ones_v, cnt_sh.at[didx(t)], csem).wait()

    rs = src_v.at[pl.ds(NFC * CHUNK, REM)]
    rd = dst_v.at[pl.ds(NFC * CHUNK, REM)]
    pltpu.sync_copy(y_hbm.at[rs], rows16_v)
    pltpu.sync_copy(rows16_v, agg_sh.at[rd], add=True)
    pltpu.sync_copy(ones_v.at[pl.ds(0, REM)], cnt_sh.at[rd], add=True)

    plsc.subcore_barrier()

    pltpu.sync_copy(agg_sh.at[pl.ds(s * RPT, RPT)],
                    agg_out.at[c].at[pl.ds(s * RPT, RPT)])
    pltpu.sync_copy(cnt_sh.at[pl.ds(s * RPT, RPT)],
                    cnt_out.at[c].at[pl.ds(s * RPT, RPT)])


_sc_aggregate = pl.kernel(
    _sc_body,
    out_type=[
        jax.ShapeDtypeStruct((NC, NP, H), jnp.float32),
        jax.ShapeDtypeStruct((NC, NP, CW), jnp.float32),
    ],
    mesh=plsc.VectorSubcoreMesh(core_axis_name="c", subcore_axis_name="s"),
    compiler_params=pltpu.CompilerParams(use_tc_tiling_on_sc=False),
    scratch_types=[
        pltpu.VMEM_SHARED((NP, H), jnp.float32),
        pltpu.VMEM_SHARED((NP, CW), jnp.float32),
        pltpu.VMEM((EPW,), jnp.int32),
        pltpu.VMEM((EPW,), jnp.int32),
        pltpu.VMEM((NBUF, CHUNK, H), jnp.float32),
        pltpu.VMEM((CHUNK, CW), jnp.float32),
        pltpu.VMEM((REM, H), jnp.float32),
        pltpu.SemaphoreType.DMA,
        pltpu.SemaphoreType.DMA,
        pltpu.SemaphoreType.DMA,
    ],
)


RPW = NP // NW


def _fin_sc_body(agg_hbm, cnt_hbm, z_hbm, out_hbm,
                 a0_v, a1_v, c0_v, c1_v, z_v, o_v):
    c = lax.axis_index("c")
    s = lax.axis_index("s")
    base = (c * NS + s) * RPW

    pltpu.sync_copy(agg_hbm.at[0].at[pl.ds(base, RPW)], a0_v)
    pltpu.sync_copy(agg_hbm.at[1].at[pl.ds(base, RPW)], a1_v)
    pltpu.sync_copy(cnt_hbm.at[0].at[pl.ds(base, RPW)], c0_v)
    pltpu.sync_copy(cnt_hbm.at[1].at[pl.ds(base, RPW)], c1_v)
    pltpu.sync_copy(z_hbm.at[pl.ds(base, RPW)], z_v)

    def row_step(r, carry):
        inv = 1.0 / jnp.maximum(c0_v[r] + c1_v[r], 1.0)
        for k in range(H // 16):
            sl = pl.ds(k * 16, 16)
            o_v[r, sl] = (a0_v[r, sl] + a1_v[r, sl]) * inv + z_v[r, sl]
        return carry

    lax.fori_loop(0, RPW, row_step, 0)

    pltpu.sync_copy(o_v, out_hbm.at[pl.ds(base, RPW)])


_sc_finish = pl.kernel(
    _fin_sc_body,
    out_type=jax.ShapeDtypeStruct((NP, H), jnp.float32),
    mesh=plsc.VectorSubcoreMesh(core_axis_name="c", subcore_axis_name="s"),
    compiler_params=pltpu.CompilerParams(use_tc_tiling_on_sc=False),
    scratch_types=[
        pltpu.VMEM((RPW, H), jnp.float32),
        pltpu.VMEM((RPW, H), jnp.float32),
        pltpu.VMEM((RPW, CW), jnp.float32),
        pltpu.VMEM((RPW, CW), jnp.float32),
        pltpu.VMEM((RPW, H), jnp.float32),
        pltpu.VMEM((RPW, H), jnp.float32),
    ],
)


def kernel(x, edge_index, W_l, W_r, b_l):
    y, z = _dual_matmul(x, W_l, W_r, b_l.reshape(1, H))

    za = jnp.zeros((NP, H), jnp.float32)
    zc = jnp.zeros((NP, CW), jnp.float32)
    ones = jnp.ones((CHUNK, CW), jnp.float32)

    agg_p, cnt_p = _sc_aggregate(y, edge_index.astype(jnp.int32), za, zc, ones)

    out_full = _sc_finish(agg_p, cnt_p, z)
    return out_full[:N]


# speedup vs baseline: 1.1134x; 1.0618x over previous
import functools

import jax
import jax.numpy as jnp
from jax import lax
from jax.experimental import pallas as pl
from jax.experimental.pallas import tpu as pltpu, tpu_sc as plsc

N = 10000
E = 320000
F_IN = 128
H = 64

NC = 2
NS = 16
NW = NC * NS
CHUNK = 128
EPW = E // NW
NFC = EPW // CHUNK
REM = EPW - NFC * CHUNK
NP = 10112
RPT = NP // NS
NBUF = 6
LOOKAHEAD = NBUF - 2
CW = 16


def _mm2_body(x_ref, wl_ref, wr_ref, b_ref, y_ref, z_ref):
    xb = x_ref[...]
    dn = (((1,), (1,)), ((), ()))
    y_ref[...] = lax.dot_general(xb, wl_ref[...], dn,
                                 preferred_element_type=jnp.float32)
    z_ref[...] = lax.dot_general(xb, wr_ref[...], dn,
                                 preferred_element_type=jnp.float32) + b_ref[...]


def _dual_matmul(x, wl, wr, b_row):
    blk = 1000
    grid = N // blk
    return pl.pallas_call(
        _mm2_body,
        grid=(grid,),
        in_specs=[
            pl.BlockSpec((blk, F_IN), lambda i: (i, 0)),
            pl.BlockSpec((H, F_IN), lambda i: (0, 0)),
            pl.BlockSpec((H, F_IN), lambda i: (0, 0)),
            pl.BlockSpec((1, H), lambda i: (0, 0)),
        ],
        out_specs=[
            pl.BlockSpec((blk, H), lambda i: (i, 0)),
            pl.BlockSpec((blk, H), lambda i: (i, 0)),
        ],
        out_shape=[
            jax.ShapeDtypeStruct((NP, H), jnp.float32),
            jax.ShapeDtypeStruct((NP, H), jnp.float32),
        ],
    )(x, wl, wr, b_row)


def _sc_body(y_hbm, edge_hbm, za_hbm, zc_hbm, ones_hbm,
             agg_out, cnt_out,
             agg_sh, cnt_sh,
             src_v, dst_v, rows_v, ones_v, rows16_v,
             gsem, ssem, csem):
    c = lax.axis_index("c")
    s = lax.axis_index("s")
    w = c * NS + s

    pltpu.sync_copy(za_hbm.at[pl.ds(s * RPT, RPT)], agg_sh.at[pl.ds(s * RPT, RPT)])
    pltpu.sync_copy(zc_hbm.at[pl.ds(s * RPT, RPT)], cnt_sh.at[pl.ds(s * RPT, RPT)])

    pltpu.sync_copy(edge_hbm.at[0].at[pl.ds(w * EPW, EPW)], src_v)
    pltpu.sync_copy(edge_hbm.at[1].at[pl.ds(w * EPW, EPW)], dst_v)
    pltpu.sync_copy(ones_hbm, ones_v)

    plsc.subcore_barrier()

    def sidx(j):
        return src_v.at[pl.ds(j * CHUNK, CHUNK)]

    def didx(j):
        return dst_v.at[pl.ds(j * CHUNK, CHUNK)]

    for p in range(LOOKAHEAD):
        pltpu.async_copy(y_hbm.at[sidx(p)], rows_v.at[p], gsem)

    def chunk_step(j, carry):
        b = lax.rem(j, NBUF)

        pltpu.make_async_copy(y_hbm.at[sidx(j)], rows_v.at[b], gsem).wait()

        @pl.when(j >= 2)
        def _wait_prev_scatter():
            pltpu.make_async_copy(rows_v.at[b], agg_sh.at[didx(j)],
                                  ssem).wait()
            pltpu.make_async_copy(ones_v, cnt_sh.at[didx(j)], csem).wait()

        @pl.when(j + LOOKAHEAD < NFC)
        def _prefetch_next():
            pltpu.async_copy(y_hbm.at[sidx(j + LOOKAHEAD)],
                             rows_v.at[lax.rem(j + LOOKAHEAD, NBUF)], gsem)

        pltpu.async_copy(rows_v.at[b], agg_sh.at[didx(j)], ssem, add=True)
        pltpu.async_copy(ones_v, cnt_sh.at[didx(j)], csem, add=True)
        return carry

    lax.fori_loop(0, NFC, chunk_step, 0)

    for t in (NFC - 2, NFC - 1):
        pltpu.make_async_copy(rows_v.at[lax.rem(t, NBUF)],
                              agg_sh.at[didx(t)], ssem).wait()
        pltpu.make_async_copy(ones_v, cnt_sh.at[didx(t)], csem).wait()

    rs = src_v.at[pl.ds(NFC * CHUNK, REM)]
    rd = dst_v.at[pl.ds(NFC * CHUNK, REM)]
    pltpu.sync_copy(y_hbm.at[rs], rows16_v)
    pltpu.sync_copy(rows16_v, agg_sh.at[rd], add=True)
    pltpu.sync_copy(ones_v.at[pl.ds(0, REM)], cnt_sh.at[rd], add=True)

    plsc.subcore_barrier()

    pltpu.sync_copy(agg_sh.at[pl.ds(s * RPT, RPT)],
                    agg_out.at[c].at[pl.ds(s * RPT, RPT)])
    pltpu.sync_copy(cnt_sh.at[pl.ds(s * RPT, RPT)],
                    cnt_out.at[c].at[pl.ds(s * RPT, RPT)])


_sc_aggregate = pl.kernel(
    _sc_body,
    out_type=[
        jax.ShapeDtypeStruct((NC, NP, H), jnp.float32),
        jax.ShapeDtypeStruct((NC, NP, CW), jnp.float32),
    ],
    mesh=plsc.VectorSubcoreMesh(core_axis_name="c", subcore_axis_name="s"),
    compiler_params=pltpu.CompilerParams(use_tc_tiling_on_sc=False),
    scratch_types=[
        pltpu.VMEM_SHARED((NP, H), jnp.float32),
        pltpu.VMEM_SHARED((NP, CW), jnp.float32),
        pltpu.VMEM((EPW,), jnp.int32),
        pltpu.VMEM((EPW,), jnp.int32),
        pltpu.VMEM((NBUF, CHUNK, H), jnp.float32),
        pltpu.VMEM((CHUNK, CW), jnp.float32),
        pltpu.VMEM((REM, H), jnp.float32),
        pltpu.SemaphoreType.DMA,
        pltpu.SemaphoreType.DMA,
        pltpu.SemaphoreType.DMA,
    ],
)


RPB = 313


def _fin_sc_body(agg_hbm, cnt_hbm, z_hbm, out_hbm,
                 a0_v, a1_v, c0_v, c1_v, z_v, o_v, dsem):
    c = lax.axis_index("c")
    s = lax.axis_index("s")
    w = c * NS + s
    base = jnp.minimum(w * RPB, N - RPB)

    pltpu.async_copy(agg_hbm.at[0].at[pl.ds(base, RPB)], a0_v, dsem)
    pltpu.async_copy(agg_hbm.at[1].at[pl.ds(base, RPB)], a1_v, dsem)
    pltpu.async_copy(cnt_hbm.at[0].at[pl.ds(base, RPB)], c0_v, dsem)
    pltpu.async_copy(cnt_hbm.at[1].at[pl.ds(base, RPB)], c1_v, dsem)
    pltpu.async_copy(z_hbm.at[pl.ds(base, RPB)], z_v, dsem)
    pltpu.make_async_copy(agg_hbm.at[0].at[pl.ds(base, RPB)], a0_v, dsem).wait()
    pltpu.make_async_copy(agg_hbm.at[1].at[pl.ds(base, RPB)], a1_v, dsem).wait()
    pltpu.make_async_copy(cnt_hbm.at[0].at[pl.ds(base, RPB)], c0_v, dsem).wait()
    pltpu.make_async_copy(cnt_hbm.at[1].at[pl.ds(base, RPB)], c1_v, dsem).wait()
    pltpu.make_async_copy(z_hbm.at[pl.ds(base, RPB)], z_v, dsem).wait()

    def row_step(r, carry):
        inv = 1.0 / jnp.maximum(c0_v[r] + c1_v[r], 1.0)
        for k in range(H // 16):
            sl = pl.ds(k * 16, 16)
            o_v[r, sl] = (a0_v[r, sl] + a1_v[r, sl]) * inv + z_v[r, sl]
        return carry

    lax.fori_loop(0, RPB, row_step, 0)

    pltpu.sync_copy(o_v, out_hbm.at[pl.ds(base, RPB)])


_sc_finish = pl.kernel(
    _fin_sc_body,
    out_type=jax.ShapeDtypeStruct((N, H), jnp.float32),
    mesh=plsc.VectorSubcoreMesh(core_axis_name="c", subcore_axis_name="s"),
    compiler_params=pltpu.CompilerParams(use_tc_tiling_on_sc=False),
    scratch_types=[
        pltpu.VMEM((RPB, H), jnp.float32),
        pltpu.VMEM((RPB, H), jnp.float32),
        pltpu.VMEM((RPB, CW), jnp.float32),
        pltpu.VMEM((RPB, CW), jnp.float32),
        pltpu.VMEM((RPB, H), jnp.float32),
        pltpu.VMEM((RPB, H), jnp.float32),
        pltpu.SemaphoreType.DMA,
    ],
)


def kernel(x, edge_index, W_l, W_r, b_l):
    y, z = _dual_matmul(x, W_l, W_r, b_l.reshape(1, H))

    za = jnp.zeros((NP, H), jnp.float32)
    zc = jnp.zeros((NP, CW), jnp.float32)
    ones = jnp.ones((CHUNK, CW), jnp.float32)

    agg_p, cnt_p = _sc_aggregate(y, edge_index.astype(jnp.int32), za, zc, ones)

    return _sc_finish(agg_p, cnt_p, z)


# speedup vs baseline: 1.1867x; 1.0658x over previous
import functools

import jax
import jax.numpy as jnp
from jax import lax
from jax.experimental import pallas as pl
from jax.experimental.pallas import tpu as pltpu, tpu_sc as plsc

N = 10000
E = 320000
F_IN = 128
H = 64

NC = 2
NS = 16
NW = NC * NS
CHUNK = 128
EPW = E // NW
NFC = EPW // CHUNK
REM = EPW - NFC * CHUNK
NP = 10112
RPT = NP // NS
NBUF = 6
LOOKAHEAD = NBUF - 2
CW = 16


def _mm2_body(x_ref, wl_ref, wr_ref, b_ref, y_ref, z_ref):
    xb = x_ref[...]
    dn = (((1,), (1,)), ((), ()))
    y_ref[...] = lax.dot_general(xb, wl_ref[...], dn,
                                 preferred_element_type=jnp.float32)
    z_ref[...] = lax.dot_general(xb, wr_ref[...], dn,
                                 preferred_element_type=jnp.float32) + b_ref[...]


def _dual_matmul(x, wl, wr, b_row):
    blk = 1000
    grid = N // blk
    return pl.pallas_call(
        _mm2_body,
        grid=(grid,),
        in_specs=[
            pl.BlockSpec((blk, F_IN), lambda i: (i, 0)),
            pl.BlockSpec((H, F_IN), lambda i: (0, 0)),
            pl.BlockSpec((H, F_IN), lambda i: (0, 0)),
            pl.BlockSpec((1, H), lambda i: (0, 0)),
        ],
        out_specs=[
            pl.BlockSpec((blk, H), lambda i: (i, 0)),
            pl.BlockSpec((blk, H), lambda i: (i, 0)),
        ],
        out_shape=[
            jax.ShapeDtypeStruct((NP, H), jnp.float32),
            jax.ShapeDtypeStruct((NP, H), jnp.float32),
        ],
    )(x, wl, wr, b_row)


def _sc_body(y_hbm, edge_hbm,
             agg_out, cnt_out,
             agg_sh, cnt_sh,
             src_v, dst_v, rows_v, ones_v, rows16_v,
             gsem, ssem, csem):
    c = lax.axis_index("c")
    s = lax.axis_index("s")
    w = c * NS + s

    pltpu.async_copy(edge_hbm.at[0].at[pl.ds(w * EPW, EPW)], src_v, gsem)
    pltpu.async_copy(edge_hbm.at[1].at[pl.ds(w * EPW, EPW)], dst_v, gsem)

    def zero_init(j, carry):
        for k in range(H // 16):
            rows_v[0, j, pl.ds(k * 16, 16)] = jnp.zeros((16,), jnp.float32)
        ones_v[j] = jnp.zeros((16,), jnp.float32)
        return carry

    lax.fori_loop(0, CHUNK, zero_init, 0)

    for off, n in ((0, 128), (128, 128), (256, 128), (384, 128), (512, 120)):
        pltpu.sync_copy(rows_v.at[0].at[pl.ds(0, n)],
                        agg_sh.at[pl.ds(s * RPT + off, n)])
        pltpu.sync_copy(ones_v.at[pl.ds(0, n)],
                        cnt_sh.at[pl.ds(s * RPT + off, n)])

    def ones_init(j, carry):
        ones_v[j] = jnp.ones((16,), jnp.float32)
        return carry

    lax.fori_loop(0, CHUNK, ones_init, 0)

    pltpu.make_async_copy(edge_hbm.at[0].at[pl.ds(w * EPW, EPW)], src_v,
                          gsem).wait()
    pltpu.make_async_copy(edge_hbm.at[1].at[pl.ds(w * EPW, EPW)], dst_v,
                          gsem).wait()

    plsc.subcore_barrier()

    def sidx(j):
        return src_v.at[pl.ds(j * CHUNK, CHUNK)]

    def didx(j):
        return dst_v.at[pl.ds(j * CHUNK, CHUNK)]

    for p in range(LOOKAHEAD):
        pltpu.async_copy(y_hbm.at[sidx(p)], rows_v.at[p], gsem)

    def chunk_step(j, carry):
        b = lax.rem(j, NBUF)

        pltpu.make_async_copy(y_hbm.at[sidx(j)], rows_v.at[b], gsem).wait()

        @pl.when(j >= 2)
        def _wait_prev_scatter():
            pltpu.make_async_copy(rows_v.at[b], agg_sh.at[didx(j)],
                                  ssem).wait()
            pltpu.make_async_copy(ones_v, cnt_sh.at[didx(j)], csem).wait()

        @pl.when(j + LOOKAHEAD < NFC)
        def _prefetch_next():
            pltpu.async_copy(y_hbm.at[sidx(j + LOOKAHEAD)],
                             rows_v.at[lax.rem(j + LOOKAHEAD, NBUF)], gsem)

        pltpu.async_copy(rows_v.at[b], agg_sh.at[didx(j)], ssem, add=True)
        pltpu.async_copy(ones_v, cnt_sh.at[didx(j)], csem, add=True)
        return carry

    lax.fori_loop(0, NFC, chunk_step, 0)

    for t in (NFC - 2, NFC - 1):
        pltpu.make_async_copy(rows_v.at[lax.rem(t, NBUF)],
                              agg_sh.at[didx(t)], ssem).wait()
        pltpu.make_async_copy(ones_v, cnt_sh.at[didx(t)], csem).wait()

    rs = src_v.at[pl.ds(NFC * CHUNK, REM)]
    rd = dst_v.at[pl.ds(NFC * CHUNK, REM)]
    pltpu.sync_copy(y_hbm.at[rs], rows16_v)
    pltpu.sync_copy(rows16_v, agg_sh.at[rd], add=True)
    pltpu.sync_copy(ones_v.at[pl.ds(0, REM)], cnt_sh.at[rd], add=True)

    plsc.subcore_barrier()

    pltpu.sync_copy(agg_sh.at[pl.ds(s * RPT, RPT)],
                    agg_out.at[c].at[pl.ds(s * RPT, RPT)])
    pltpu.sync_copy(cnt_sh.at[pl.ds(s * RPT, RPT)],
                    cnt_out.at[c].at[pl.ds(s * RPT, RPT)])


_sc_aggregate = pl.kernel(
    _sc_body,
    out_type=[
        jax.ShapeDtypeStruct((NC, NP, H), jnp.float32),
        jax.ShapeDtypeStruct((NC, NP, CW), jnp.float32),
    ],
    mesh=plsc.VectorSubcoreMesh(core_axis_name="c", subcore_axis_name="s"),
    compiler_params=pltpu.CompilerParams(use_tc_tiling_on_sc=False),
    scratch_types=[
        pltpu.VMEM_SHARED((NP, H), jnp.float32),
        pltpu.VMEM_SHARED((NP, CW), jnp.float32),
        pltpu.VMEM((EPW,), jnp.int32),
        pltpu.VMEM((EPW,), jnp.int32),
        pltpu.VMEM((NBUF, CHUNK, H), jnp.float32),
        pltpu.VMEM((CHUNK, CW), jnp.float32),
        pltpu.VMEM((REM, H), jnp.float32),
        pltpu.SemaphoreType.DMA,
        pltpu.SemaphoreType.DMA,
        pltpu.SemaphoreType.DMA,
    ],
)


RPB = 313


def _fin_sc_body(agg_hbm, cnt_hbm, z_hbm, out_hbm,
                 a0_v, a1_v, c0_v, c1_v, z_v, o_v, dsem):
    c = lax.axis_index("c")
    s = lax.axis_index("s")
    w = c * NS + s
    base = jnp.minimum(w * RPB, N - RPB)

    pltpu.async_copy(agg_hbm.at[0].at[pl.ds(base, RPB)], a0_v, dsem)
    pltpu.async_copy(agg_hbm.at[1].at[pl.ds(base, RPB)], a1_v, dsem)
    pltpu.async_copy(cnt_hbm.at[0].at[pl.ds(base, RPB)], c0_v, dsem)
    pltpu.async_copy(cnt_hbm.at[1].at[pl.ds(base, RPB)], c1_v, dsem)
    pltpu.async_copy(z_hbm.at[pl.ds(base, RPB)], z_v, dsem)
    pltpu.make_async_copy(agg_hbm.at[0].at[pl.ds(base, RPB)], a0_v, dsem).wait()
    pltpu.make_async_copy(agg_hbm.at[1].at[pl.ds(base, RPB)], a1_v, dsem).wait()
    pltpu.make_async_copy(cnt_hbm.at[0].at[pl.ds(base, RPB)], c0_v, dsem).wait()
    pltpu.make_async_copy(cnt_hbm.at[1].at[pl.ds(base, RPB)], c1_v, dsem).wait()
    pltpu.make_async_copy(z_hbm.at[pl.ds(base, RPB)], z_v, dsem).wait()

    def row_step(r, carry):
        inv = 1.0 / jnp.maximum(c0_v[r] + c1_v[r], 1.0)
        for k in range(H // 16):
            sl = pl.ds(k * 16, 16)
            o_v[r, sl] = (a0_v[r, sl] + a1_v[r, sl]) * inv + z_v[r, sl]
        return carry

    lax.fori_loop(0, RPB, row_step, 0)

    pltpu.sync_copy(o_v, out_hbm.at[pl.ds(base, RPB)])


_sc_finish = pl.kernel(
    _fin_sc_body,
    out_type=jax.ShapeDtypeStruct((N, H), jnp.float32),
    mesh=plsc.VectorSubcoreMesh(core_axis_name="c", subcore_axis_name="s"),
    compiler_params=pltpu.CompilerParams(use_tc_tiling_on_sc=False),
    scratch_types=[
        pltpu.VMEM((RPB, H), jnp.float32),
        pltpu.VMEM((RPB, H), jnp.float32),
        pltpu.VMEM((RPB, CW), jnp.float32),
        pltpu.VMEM((RPB, CW), jnp.float32),
        pltpu.VMEM((RPB, H), jnp.float32),
        pltpu.VMEM((RPB, H), jnp.float32),
        pltpu.SemaphoreType.DMA,
    ],
)


def kernel(x, edge_index, W_l, W_r, b_l):
    y, z = _dual_matmul(x, W_l, W_r, b_l.reshape(1, H))

    agg_p, cnt_p = _sc_aggregate(y, edge_index.astype(jnp.int32))

    return _sc_finish(agg_p, cnt_p, z)
